# Initial kernel scaffold; baseline (speedup 1.0000x reference)
import jax
import jax.numpy as jnp
from jax.experimental import pallas as pl


def kernel(features_0, features_1, type_mask, mp_m0, mp_m1, mp_d0, mp_d1, dst_m0, dst_m1, dst_d0, dst_d1, target_m, target_d, params):
    raise NotImplementedError("write your pallas kernel here")


# speedup vs baseline: 4.7485x; 4.7485x over previous
import functools

import jax
import jax.numpy as jnp
from jax.experimental import pallas as pl

H = 8
D = 64
HD = H * D
E_BLK = 1000


def _edge_body(med_ref, a1_ref, w_ref, b_ref, m_ref, r_ref, wef_ref, ea_ref):
    z = jnp.dot(med_ref[...], w_ref[...], preferred_element_type=jnp.float32)
    eft = jnp.tanh(z + b_ref[...])
    a2 = jnp.dot(eft, m_ref[...], preferred_element_type=jnp.float32)
    a = a1_ref[...] + a2
    a = jnp.where(a >= 0, a, 0.2 * a)
    ea = jnp.exp(a)
    ea_ref[...] = ea
    wef_ref[...] = eft * jnp.dot(ea, r_ref[...], preferred_element_type=jnp.float32)


def _edge_pass(med, a1, w, b, m, r):
    e = med.shape[0]
    grid = (e // E_BLK,)
    return pl.pallas_call(
        _edge_body,
        grid=grid,
        in_specs=[
            pl.BlockSpec((E_BLK, D), lambda i: (i, 0)),
            pl.BlockSpec((E_BLK, H), lambda i: (i, 0)),
            pl.BlockSpec((D, HD), lambda i: (0, 0)),
            pl.BlockSpec((1, HD), lambda i: (0, 0)),
            pl.BlockSpec((HD, H), lambda i: (0, 0)),
            pl.BlockSpec((H, HD), lambda i: (0, 0)),
        ],
        out_specs=[
            pl.BlockSpec((E_BLK, HD), lambda i: (i, 0)),
            pl.BlockSpec((E_BLK, H), lambda i: (i, 0)),
        ],
        out_shape=[
            jax.ShapeDtypeStruct((e, HD), jnp.float32),
            jax.ShapeDtypeStruct((e, H), jnp.float32),
        ],
    )(med, a1, w, b, m, r)


def _metapath(feats, proj, idx, dst, w, b, m, r, t):
    med = jnp.take(feats, idx, axis=0).mean(axis=1)
    a1 = jnp.take(proj, idx[:, -1], axis=0)
    wef, ea = _edge_pass(med, a1, w, b, m, r)
    denom = jax.ops.segment_sum(ea, dst, num_segments=t)
    ftu = jax.ops.segment_sum(wef, dst, num_segments=t)
    scale = 1.0 / (denom + 1e-9)
    ft = ftu * jnp.repeat(scale, D, axis=1)
    return jax.nn.elu(ft)


def kernel(features_0, features_1, type_mask, mp_m0, mp_m1, mp_d0, mp_d1,
           dst_m0, dst_m1, dst_d0, dst_d1, target_m, target_d, params):
    p = params
    t = features_0.shape[0]
    tf0 = features_0 @ p['fc0_W'].T + p['fc0_b']
    tf1 = features_1 @ p['fc1_W'].T + p['fc1_b']
    feats = jnp.concatenate([tf0, tf1], axis=0)

    eye = jnp.eye(H, dtype=jnp.float32)
    rep = jnp.repeat(eye, D, axis=1)

    outs = {}
    for nt, mps, dsts in (('m', (mp_m0, mp_m1), (dst_m0, dst_m1)),
                          ('d', (mp_d0, mp_d1), (dst_d0, dst_d1))):
        res = []
        for i in range(2):
            w = p[nt + '_rnn_W' + str(i)].T
            b = p[nt + '_rnn_b' + str(i)].reshape(1, HD)
            attn2 = p[nt + '_attn2_' + str(i)]
            m = (attn2[:, :, None] * eye[:, None, :]).reshape(HD, H)
            proj = feats @ p[nt + '_attn1_W' + str(i)].T
            res.append(_metapath(feats, proj, mps[i], dsts[i], w, b, m, rep, t))
        betas = []
        for out in res:
            s = jnp.tanh(out @ p[nt + '_sem_fc1_W'].T + p[nt + '_sem_fc1_b'])
            betas.append(jnp.mean(s @ p[nt + '_sem_fc2_W'].T))
        beta = jax.nn.softmax(jnp.stack(betas))
        outs[nt] = beta[0] * res[0] + beta[1] * res[1]

    lm = outs['m'] @ p['m_out_W'].T + p['m_out_b']
    ld = outs['d'] @ p['d_out_W'].T + p['d_out_b']
    return (lm, ld, outs['m'], outs['d'])


# speedup vs baseline: 11.8686x; 2.4994x over previous
import functools

import jax
import jax.numpy as jnp
from jax import lax
from jax.experimental import pallas as pl
from jax.experimental.pallas import tpu as pltpu
from jax.experimental.pallas import tpu_sc as plsc

H = 8
D = 64
HD = H * D
E_BLK = 1184

NC = 2
NS = 16
NW = NC * NS
CHS = 296


def _sc_gather_body(feats_hbm, idx_hbm, p0, p1, p2, p3,
                    m0, m1, m2, m3, a0, a1, a2, a3,
                    b0, b1, b2, r16, ib0, ib1, ib2, sem,
                    *, ch, k, e_pad):
    wid = lax.axis_index("s") * NC + lax.axis_index("c")
    projs = (p0, p1, p2, p3)
    meds = (m0, m1, m2, m3)
    a1s = (a0, a1, a2, a3)
    ibs = (ib0, ib1, ib2)
    for mp in range(4):
        def jbody(j, _, mp=mp):
            row0 = wid * ch + j * CHS
            for l in range(3):
                pltpu.sync_copy(
                    idx_hbm.at[pl.ds((3 * mp + l) * e_pad + row0, CHS)],
                    ibs[l])
            c0 = pltpu.async_copy(feats_hbm.at[ib0], b0, sem)
            c1 = pltpu.async_copy(feats_hbm.at[ib1], b1, sem)
            c2 = pltpu.async_copy(feats_hbm.at[ib2], b2, sem)
            c3 = pltpu.async_copy(projs[mp].at[ib2], r16, sem)
            c0.wait()
            c1.wait()
            c2.wait()
            c3.wait()

            def rbody(r, _):
                for c in range(4):
                    sl = pl.ds(c * 16, 16)
                    b0[r, sl] = b0[r, sl] + b1[r, sl] + b2[r, sl]
                return 0

            lax.fori_loop(0, CHS, rbody, 0)
            pltpu.sync_copy(b0, meds[mp].at[pl.ds(row0, CHS)])
            pltpu.sync_copy(r16, a1s[mp].at[pl.ds(row0, CHS)])
            return 0

        lax.fori_loop(0, k, jbody, 0)


def _sc_gather(feats, projs, idxs, e_pad):
    ch = e_pad // NW
    k = ch // CHS
    n = feats.shape[0]
    mesh = plsc.VectorSubcoreMesh(core_axis_name="c", subcore_axis_name="s")
    fn = pl.kernel(
        functools.partial(_sc_gather_body, ch=ch, k=k, e_pad=e_pad),
        out_type=[jax.ShapeDtypeStruct((e_pad, D), jnp.float32)] * 4
                 + [jax.ShapeDtypeStruct((e_pad, 16), jnp.float32)] * 4,
        mesh=mesh,
        compiler_params=pltpu.CompilerParams(use_tc_tiling_on_sc=False),
        scratch_types=[pltpu.VMEM((CHS, D), jnp.float32)] * 3
                      + [pltpu.VMEM((CHS, 16), jnp.float32)]
                      + [pltpu.VMEM((CHS,), jnp.int32)] * 3
                      + [pltpu.SemaphoreType.DMA],
    )
    return fn(feats, idxs, *projs)


def _edge_body(med_ref, a1_ref, w_ref, b_ref, m_ref, r_ref, wef_ref, ea_ref):
    z = jnp.dot(med_ref[...], w_ref[...], preferred_element_type=jnp.float32)
    eft = jnp.tanh(z + b_ref[...])
    a2 = jnp.dot(eft, m_ref[...], preferred_element_type=jnp.float32)
    a = a1_ref[...][:, :H] + a2
    a = jnp.where(a >= 0, a, 0.2 * a)
    ea = jnp.exp(a)
    ea_ref[...] = ea
    wef_ref[...] = eft * jnp.dot(ea, r_ref[...], preferred_element_type=jnp.float32)


def _edge_pass(med, a1, w, b, m, r):
    e = med.shape[0]
    grid = (e // E_BLK,)
    return pl.pallas_call(
        _edge_body,
        grid=grid,
        in_specs=[
            pl.BlockSpec((E_BLK, D), lambda i: (i, 0)),
            pl.BlockSpec((E_BLK, 16), lambda i: (i, 0)),
            pl.BlockSpec((D, HD), lambda i: (0, 0)),
            pl.BlockSpec((1, HD), lambda i: (0, 0)),
            pl.BlockSpec((HD, H), lambda i: (0, 0)),
            pl.BlockSpec((H, HD), lambda i: (0, 0)),
        ],
        out_specs=[
            pl.BlockSpec((E_BLK, HD), lambda i: (i, 0)),
            pl.BlockSpec((E_BLK, H), lambda i: (i, 0)),
        ],
        out_shape=[
            jax.ShapeDtypeStruct((e, HD), jnp.float32),
            jax.ShapeDtypeStruct((e, H), jnp.float32),
        ],
    )(med, a1, w, b, m, r)


def kernel(features_0, features_1, type_mask, mp_m0, mp_m1, mp_d0, mp_d1,
           dst_m0, dst_m1, dst_d0, dst_d1, target_m, target_d, params):
    p = params
    t = features_0.shape[0]
    e = mp_m0.shape[0]
    chunk = NW * CHS
    e_pad = ((e + chunk - 1) // chunk) * chunk

    tf0 = features_0 @ p['fc0_W'].T + p['fc0_b']
    tf1 = features_1 @ p['fc1_W'].T + p['fc1_b']
    feats = jnp.concatenate([tf0, tf1], axis=0)

    eye = jnp.eye(H, dtype=jnp.float32)
    rep = jnp.repeat(eye, D, axis=1)

    mps = {'m': (mp_m0, mp_m1), 'd': (mp_d0, mp_d1)}
    dsts = {'m': (dst_m0, dst_m1), 'd': (dst_d0, dst_d1)}
    keys = [('m', 0), ('m', 1), ('d', 0), ('d', 1)]

    pad_i = jnp.zeros((e_pad - e, 3), jnp.int32)
    idxs = jnp.concatenate(
        [jnp.concatenate([mps[nt][i], pad_i], axis=0).T for nt, i in keys],
        axis=0).reshape(-1)
    projs = []
    for nt, i in keys:
        wp = jnp.zeros((D, 16), jnp.float32).at[:, :H].set(
            p[nt + '_attn1_W' + str(i)].T)
        projs.append(feats @ wp)

    meds0, meds1, meds2, meds3, a1e0, a1e1, a1e2, a1e3 = _sc_gather(
        feats, projs, idxs, e_pad)
    meds = {('m', 0): meds0, ('m', 1): meds1, ('d', 0): meds2, ('d', 1): meds3}
    a1es = {('m', 0): a1e0, ('m', 1): a1e1, ('d', 0): a1e2, ('d', 1): a1e3}

    pad_d = jnp.full((e_pad - e,), t, jnp.int32)
    outs = {}
    for nt in ('m', 'd'):
        res = []
        for i in range(2):
            w = p[nt + '_rnn_W' + str(i)].T / 3.0
            b = p[nt + '_rnn_b' + str(i)].reshape(1, HD)
            attn2 = p[nt + '_attn2_' + str(i)]
            m = (attn2[:, :, None] * eye[:, None, :]).reshape(HD, H)
            wef, ea = _edge_pass(meds[(nt, i)], a1es[(nt, i)], w, b, m, rep)
            dst = jnp.concatenate([dsts[nt][i], pad_d])
            denom = jax.ops.segment_sum(ea, dst, num_segments=t)
            ftu = jax.ops.segment_sum(wef, dst, num_segments=t)
            scale = 1.0 / (denom + 1e-9)
            ft = ftu * jnp.repeat(scale, D, axis=1)
            res.append(jax.nn.elu(ft))
        betas = []
        for out in res:
            s = jnp.tanh(out @ p[nt + '_sem_fc1_W'].T + p[nt + '_sem_fc1_b'])
            betas.append(jnp.mean(s @ p[nt + '_sem_fc2_W'].T))
        beta = jax.nn.softmax(jnp.stack(betas))
        outs[nt] = beta[0] * res[0] + beta[1] * res[1]

    lm = outs['m'] @ p['m_out_W'].T + p['m_out_b']
    ld = outs['d'] @ p['d_out_W'].T + p['d_out_b']
    return (lm, ld, outs['m'], outs['d'])


# speedup vs baseline: 20.4836x; 1.7259x over previous
import functools

import jax
import jax.numpy as jnp
from jax import lax
from jax.experimental import pallas as pl
from jax.experimental.pallas import tpu as pltpu
from jax.experimental.pallas import tpu_sc as plsc

H = 8
D = 64
HD = H * D

NC = 2
NS = 16
NW = NC * NS
CHS = 296

CH = 512
TT = 128


def _sc_gather_body(feats_hbm, idx_hbm, p0, p1, p2, p3,
                    o0, o1, o2, o3,
                    b0, b1, b2, r16, ib0, ib1, ib2, sem,
                    *, ch, k, e_pad):
    wid = lax.axis_index("s") * NC + lax.axis_index("c")
    projs = (p0, p1, p2, p3)
    outs = (o0, o1, o2, o3)
    ibs = (ib0, ib1, ib2)
    for mp in range(4):
        def jbody(j, _, mp=mp):
            row0 = wid * ch + j * CHS
            for l in range(3):
                pltpu.sync_copy(
                    idx_hbm.at[pl.ds((3 * mp + l) * e_pad + row0, CHS)],
                    ibs[l])
            c0 = pltpu.async_copy(feats_hbm.at[ib0], b0, sem)
            c1 = pltpu.async_copy(feats_hbm.at[ib1], b1, sem)
            c2 = pltpu.async_copy(feats_hbm.at[ib2], b2, sem)
            c3 = pltpu.async_copy(projs[mp].at[ib2], r16, sem)
            c0.wait()
            c1.wait()
            c2.wait()
            c3.wait()

            def rbody(r, _):
                for c in range(4):
                    sl = pl.ds(c * 16, 16)
                    b0[r, sl] = b0[r, sl] + b1[r, sl] + b2[r, sl]
                return 0

            lax.fori_loop(0, CHS, rbody, 0)
            pltpu.sync_copy(b0, outs[mp].at[pl.ds(row0, CHS), pl.ds(0, D)])
            pltpu.sync_copy(r16, outs[mp].at[pl.ds(row0, CHS), pl.ds(D, 16)])
            return 0

        lax.fori_loop(0, k, jbody, 0)


def _sc_gather(feats, projs, idxs, e_pad):
    ch = e_pad // NW
    k = ch // CHS
    mesh = plsc.VectorSubcoreMesh(core_axis_name="c", subcore_axis_name="s")
    fn = pl.kernel(
        functools.partial(_sc_gather_body, ch=ch, k=k, e_pad=e_pad),
        out_type=[jax.ShapeDtypeStruct((e_pad, 128), jnp.float32)] * 4,
        mesh=mesh,
        compiler_params=pltpu.CompilerParams(use_tc_tiling_on_sc=False),
        scratch_types=[pltpu.VMEM((CHS, D), jnp.float32)] * 3
                      + [pltpu.VMEM((CHS, 16), jnp.float32)]
                      + [pltpu.VMEM((CHS,), jnp.int32)] * 3
                      + [pltpu.SemaphoreType.DMA],
    )
    return fn(feats, idxs, *projs)


def _seg_body(offs_ref, comb_ref, dst_ref, w_ref, b_ref, m_ref, s_ref, r_ref,
              out_ref, combb, dstb, sem_c, sem_d):
    k = pl.program_id(0)
    s = offs_ref[k]
    e_ = offs_ref[k + 1]
    c_lo = s // CH
    c_hi = (e_ + CH - 1) // CH
    acc0 = jnp.zeros((TT, HD), jnp.float32)
    acc1 = jnp.zeros((TT, H), jnp.float32)
    rows = k * TT + lax.broadcasted_iota(jnp.int32, (TT, CH), 0)

    def chunk(i, carry):
        acc0, acc1 = carry
        c = c_lo + i
        cm = pltpu.make_async_copy(comb_ref.at[pl.ds(c * CH, CH)], combb,
                                   sem_c)
        cd = pltpu.make_async_copy(dst_ref.at[pl.ds(c * CH, CH)], dstb,
                                   sem_d)
        cm.start()
        cd.start()
        cm.wait()
        cd.wait()
        comb = combb[...]
        z = jnp.dot(comb, w_ref[...], preferred_element_type=jnp.float32)
        eft = jnp.tanh(z + b_ref[...])
        a2 = jnp.dot(eft, m_ref[...], preferred_element_type=jnp.float32)
        a1 = jnp.dot(comb, s_ref[...], preferred_element_type=jnp.float32)
        a = a1 + a2
        a = jnp.where(a >= 0, a, 0.2 * a)
        ea = jnp.exp(a)
        wef = eft * jnp.dot(ea, r_ref[...], preferred_element_type=jnp.float32)
        oh = (rows == dstb[...].reshape(1, CH)).astype(jnp.float32)
        acc0 = acc0 + jnp.dot(oh, wef, preferred_element_type=jnp.float32)
        acc1 = acc1 + jnp.dot(oh, ea, preferred_element_type=jnp.float32)
        return acc0, acc1

    acc0, acc1 = lax.fori_loop(0, c_hi - c_lo, chunk, (acc0, acc1))
    scale = 1.0 / (acc1 + 1e-9)
    ft = acc0 * jnp.dot(scale, r_ref[...], preferred_element_type=jnp.float32)
    out_ref[...] = jnp.where(ft > 0, ft, jnp.exp(jnp.minimum(ft, 0.0)) - 1.0)


def _seg_pass(offs, comb, dst, w, b, m, sel, r, t_pad):
    return pl.pallas_call(
        _seg_body,
        grid=(t_pad // TT,),
        in_specs=[
            pl.BlockSpec(memory_space=pltpu.MemorySpace.SMEM),
            pl.BlockSpec(memory_space=pl.ANY),
            pl.BlockSpec(memory_space=pl.ANY),
            pl.BlockSpec((128, HD), lambda i: (0, 0)),
            pl.BlockSpec((1, HD), lambda i: (0, 0)),
            pl.BlockSpec((HD, H), lambda i: (0, 0)),
            pl.BlockSpec((128, H), lambda i: (0, 0)),
            pl.BlockSpec((H, HD), lambda i: (0, 0)),
        ],
        out_specs=pl.BlockSpec((TT, HD), lambda i: (i, 0)),
        out_shape=jax.ShapeDtypeStruct((t_pad, HD), jnp.float32),
        scratch_shapes=[
            pltpu.VMEM((CH, 128), jnp.float32),
            pltpu.VMEM((CH,), jnp.int32),
            pltpu.SemaphoreType.DMA,
            pltpu.SemaphoreType.DMA,
        ],
    )(offs, comb, dst, w, b, m, sel, r)


def kernel(features_0, features_1, type_mask, mp_m0, mp_m1, mp_d0, mp_d1,
           dst_m0, dst_m1, dst_d0, dst_d1, target_m, target_d, params):
    p = params
    t = features_0.shape[0]
    e = mp_m0.shape[0]
    chunk = NW * CHS
    e_pad = ((e + chunk - 1) // chunk) * chunk
    t_pad = ((t + TT - 1) // TT) * TT

    tf0 = features_0 @ p['fc0_W'].T + p['fc0_b']
    tf1 = features_1 @ p['fc1_W'].T + p['fc1_b']
    feats = jnp.concatenate([tf0, tf1], axis=0)

    eye = jnp.eye(H, dtype=jnp.float32)
    rep = jnp.repeat(eye, D, axis=1)
    sel = jnp.zeros((128, H), jnp.float32).at[D:D + H].set(eye)

    mps = {'m': (mp_m0, mp_m1), 'd': (mp_d0, mp_d1)}
    dsts = {'m': (dst_m0, dst_m1), 'd': (dst_d0, dst_d1)}
    keys = [('m', 0), ('m', 1), ('d', 0), ('d', 1)]

    pad_i = jnp.zeros((e_pad - e, 3), jnp.int32)
    idxs = jnp.concatenate(
        [jnp.concatenate([mps[nt][i], pad_i], axis=0).T for nt, i in keys],
        axis=0).reshape(-1)
    projs = []
    for nt, i in keys:
        wp = jnp.zeros((D, 16), jnp.float32).at[:, :H].set(
            p[nt + '_attn1_W' + str(i)].T)
        projs.append(feats @ wp)

    combs = _sc_gather(feats, projs, idxs, e_pad)
    combd = dict(zip(keys, combs))

    pad_d = jnp.full((e_pad - e,), t, jnp.int32)
    tile_starts = jnp.arange(t_pad // TT + 1, dtype=jnp.int32) * TT
    outs = {}
    for nt in ('m', 'd'):
        res = []
        for i in range(2):
            w = jnp.zeros((128, HD), jnp.float32).at[:D].set(
                p[nt + '_rnn_W' + str(i)].T / 3.0)
            b = p[nt + '_rnn_b' + str(i)].reshape(1, HD)
            attn2 = p[nt + '_attn2_' + str(i)]
            m = (attn2[:, :, None] * eye[:, None, :]).reshape(HD, H)
            dst = jnp.concatenate([dsts[nt][i], pad_d])
            offs = jnp.searchsorted(dst, tile_starts).astype(jnp.int32)
            ft = _seg_pass(offs, combd[(nt, i)], dst, w, b, m, sel, rep,
                           t_pad)
            res.append(ft[:t])
        betas = []
        for out in res:
            s = jnp.tanh(out @ p[nt + '_sem_fc1_W'].T + p[nt + '_sem_fc1_b'])
            betas.append(jnp.mean(s @ p[nt + '_sem_fc2_W'].T))
        beta = jax.nn.softmax(jnp.stack(betas))
        outs[nt] = beta[0] * res[0] + beta[1] * res[1]

    lm = outs['m'] @ p['m_out_W'].T + p['m_out_b']
    ld = outs['d'] @ p['d_out_W'].T + p['d_out_b']
    return (lm, ld, outs['m'], outs['d'])
